# Initial kernel scaffold; baseline (speedup 1.0000x reference)
import jax
import jax.numpy as jnp
from jax.experimental import pallas as pl


def kernel(inp_rows, inp_cols, inp_values, weight):
    raise NotImplementedError("write your pallas kernel here")


# speedup vs baseline: 2.3771x; 2.3771x over previous
import dataclasses
import functools

import jax
import jax.numpy as jnp
from jax import lax
from jax.experimental import pallas as pl
from jax.experimental.pallas import tpu as pltpu
from jax.experimental.pallas import tpu_sc as plsc

N = 16384
NNZ = 268435
D = 256

NC = 2
NS = 16
NW = NC * NS
L = 16

ROWS_PER_CHUNK = 256
NUM_CHUNKS = N // ROWS_PER_CHUNK
CHUNKS_PER_TILE = NUM_CHUNKS // NW
W = 64
NNZ_PAD = ((NNZ + W + 7) // 8) * 8
OFFS_PAD = 96


def _sc_spmm(rows_p, cols_p, vals_p, offs, weight):
    mesh = plsc.VectorSubcoreMesh(core_axis_name="c", subcore_axis_name="s")
    cp = pltpu.CompilerParams()
    if "needs_layout_passes" in pltpu.CompilerParams.__dataclass_fields__:
        cp = dataclasses.replace(cp, needs_layout_passes=False)

    @functools.partial(
        pl.kernel,
        compiler_params=cp,
        out_type=jax.ShapeDtypeStruct((N * D,), jnp.float32),
        mesh=mesh,
        scratch_types=[
            pltpu.VMEM((OFFS_PAD,), jnp.int32),
            pltpu.VMEM((W,), jnp.int32),
            pltpu.VMEM((W,), jnp.int32),
            pltpu.VMEM((W,), jnp.float32),
            pltpu.VMEM((W, D), jnp.float32),
            pltpu.VMEM((ROWS_PER_CHUNK * D,), jnp.float32),
            pltpu.SemaphoreType.DMA,
        ],
    )
    def sc_kernel(rows_hbm, cols_hbm, vals_hbm, offs_hbm, w_hbm, out_hbm,
                  offs_v, rows_v, cols_v, vals_v, g_v, acc_v, sem):
        wid = lax.axis_index("s") * NC + lax.axis_index("c")
        pltpu.sync_copy(offs_hbm, offs_v)
        lane = lax.broadcasted_iota(jnp.int32, (L,), 0)
        zero16 = jnp.zeros((L,), jnp.float32)

        for cc in range(CHUNKS_PER_TILE):
            c = wid * CHUNKS_PER_TILE + cc
            base_row = c * ROWS_PER_CHUNK
            ov = offs_v[pl.ds(c, L)]
            start = jnp.sum(jnp.where(lane == 0, ov, 0))
            end = jnp.sum(jnp.where(lane == 1, ov, 0))

            @pl.loop(0, ROWS_PER_CHUNK * D // L)
            def _(i):
                acc_v[pl.ds(i * L, L)] = zero16

            a0 = start - (start & 7)
            nwin = (end - a0 + W - 1) // W

            @pl.loop(0, nwin)
            def _(win):
                k0 = pl.multiple_of(a0 + win * W, 8)
                pltpu.sync_copy(rows_hbm.at[pl.ds(k0, W)], rows_v)
                pltpu.sync_copy(cols_hbm.at[pl.ds(k0, W)], cols_v)
                pltpu.sync_copy(vals_hbm.at[pl.ds(k0, W)], vals_v)
                pltpu.async_copy(w_hbm.at[cols_v], g_v, sem).wait()

                for g16 in range(W // L):
                    rv = rows_v[pl.ds(g16 * L, L)]
                    vv = vals_v[pl.ds(g16 * L, L)]
                    pos = (k0 + g16 * L) + lane
                    valid = (pos >= start) & (pos < end)
                    vv = jnp.where(valid, vv, 0.0)
                    lr = jnp.clip(rv - base_row, 0, ROWS_PER_CHUNK - 1)

                    @pl.loop(0, L)
                    def _(j):
                        sel = lane == j
                        row_j = jnp.sum(jnp.where(sel, lr, 0))
                        val_j = jnp.sum(jnp.where(sel, vv, 0.0))
                        abase = row_j * D
                        kfull = jnp.zeros((L,), jnp.int32) + (g16 * L + j)
                        for dj in range(D // L):
                            gsl = plsc.load_gather(g_v, [kfull, dj * L + lane])
                            sl = pl.ds(abase + dj * L, L)
                            acc_v[sl] = acc_v[sl] + val_j * gsl

            pltpu.sync_copy(
                acc_v, out_hbm.at[pl.ds(base_row * D, ROWS_PER_CHUNK * D)])

    return sc_kernel(rows_p, cols_p, vals_p, offs, weight)


def kernel(inp_rows, inp_cols, inp_values, weight):
    offs = jnp.searchsorted(
        inp_rows, jnp.arange(0, N + 1, ROWS_PER_CHUNK), side="left"
    ).astype(jnp.int32)
    offs = jnp.pad(offs, (0, OFFS_PAD - offs.shape[0]), constant_values=NNZ)
    pad = NNZ_PAD - NNZ
    rows_p = jnp.pad(inp_rows, (0, pad), constant_values=N - 1)
    cols_p = jnp.pad(inp_cols, (0, pad), constant_values=0)
    vals_p = jnp.pad(inp_values, (0, pad), constant_values=0.0)
    out_flat = _sc_spmm(rows_p, cols_p, vals_p, offs, weight)
    return out_flat.reshape(N, D)
